# baseline (speedup 1.0000x reference)
import functools

import jax
import jax.numpy as jnp
from jax import lax
from jax.experimental import pallas as pl
from jax.experimental.pallas import tpu as pltpu
from jax.experimental.pallas import tpu_sc as plsc

_B = 4096
_L = 200
_EMB = 64
_C0 = 128
_C1 = _L - _C0
_NBUF = 4
_G = 2

_info = plsc.get_sparse_core_info()
_NC, _NS = _info.num_cores, _info.num_subcores
_NW = _NC * _NS
_BATCHES_PER_W = _B // _NW
_NBLK = _BATCHES_PER_W // _NBUF


def _make_lookup():
    mesh = plsc.VectorSubcoreMesh(core_axis_name="c", subcore_axis_name="s")

    @functools.partial(
        pl.kernel,
        mesh=mesh,
        compiler_params=pltpu.CompilerParams(use_tc_tiling_on_sc=False),
        out_type=jax.ShapeDtypeStruct((_B * _L, _EMB), jnp.float32),
        scratch_types=(
            [pltpu.VMEM((_BATCHES_PER_W, _L), jnp.int32),
             pltpu.VMEM((_NBUF, _L, _EMB), jnp.float32)]
            + [pltpu.SemaphoreType.DMA] * (2 * _NBUF)
        ),
    )
    def lookup(ids_hbm, table_hbm, out_hbm, idx_v, rows_v, *sems):
        gsem = sems[:_NBUF]
        wsem = sems[_NBUF:]
        wid = lax.axis_index("s") * _NC + lax.axis_index("c")
        base = wid * _BATCHES_PER_W
        pltpu.sync_copy(ids_hbm.at[pl.ds(base, _BATCHES_PER_W)], idx_v)

        def start_gather(i, s):
            pltpu.async_copy(table_hbm.at[idx_v.at[i, pl.ds(0, _C0)]],
                             rows_v.at[s, pl.ds(0, _C0)], gsem[s])
            pltpu.async_copy(table_hbm.at[idx_v.at[i, pl.ds(_C0, _C1)]],
                             rows_v.at[s, pl.ds(_C0, _C1)], gsem[s])

        def wait_gather(s):
            pltpu.make_async_copy(table_hbm.at[pl.ds(0, _L)], rows_v.at[s],
                                  gsem[s]).wait()

        def start_write(i, s):
            pltpu.async_copy(rows_v.at[s],
                             out_hbm.at[pl.ds((base + i) * _L, _L)], wsem[s])

        def wait_write(i, s):
            pltpu.make_async_copy(rows_v.at[s],
                                  out_hbm.at[pl.ds((base + i) * _L, _L)],
                                  wsem[s]).wait()

        for s in range(_G):
            start_gather(s, s)

        for s in range(_NBUF):
            i = s
            wait_gather(s)
            start_write(i, s)
            ig = i + _G
            sg = ig % _NBUF
            if ig >= _NBUF:
                wait_write(ig - _NBUF, sg)
            start_gather(ig, sg)

        def block(k, carry):
            i0 = k * _NBUF
            for s in range(_NBUF):
                i = i0 + s
                wait_gather(s)
                start_write(i, s)
                ig = i + _G
                sg = (s + _G) % _NBUF
                wait_write(ig - _NBUF, sg)
                start_gather(ig, sg)
            return carry

        lax.fori_loop(1, _NBLK - 1, block, 0)

        i0 = (_NBLK - 1) * _NBUF
        for s in range(_NBUF):
            i = i0 + s
            wait_gather(s)
            start_write(i, s)
            ig = i + _G
            if ig < _BATCHES_PER_W:
                sg = (s + _G) % _NBUF
                wait_write(ig - _NBUF, sg)
                start_gather(ig, sg)

        for s in range(_NBUF):
            wait_write(i0 + s, s)

    return lookup


_lookup = _make_lookup()


def kernel(word_ids, word_emb_table):
    out = _lookup(word_ids.astype(jnp.int32), word_emb_table)
    return out.reshape(_B, _L, _EMB)


# speedup vs baseline: 1.2185x; 1.2185x over previous
import functools

import jax
import jax.numpy as jnp
from jax import lax
from jax.experimental import pallas as pl
from jax.experimental.pallas import tpu as pltpu
from jax.experimental.pallas import tpu_sc as plsc

_B = 4096
_L = 200
_EMB = 64
_W = 128
_C0 = 128
_C1 = _L - _C0
_NBUF = 4
_G = 2

_info = plsc.get_sparse_core_info()
_NC, _NS = _info.num_cores, _info.num_subcores
_NW = _NC * _NS
_BATCHES_PER_W = _B // _NW
_IDS_PER_W = _BATCHES_PER_W * _L
_NBLK = _BATCHES_PER_W // _NBUF


def _make_lookup():
    mesh = plsc.VectorSubcoreMesh(core_axis_name="c", subcore_axis_name="s")

    @functools.partial(
        pl.kernel,
        mesh=mesh,
        compiler_params=pltpu.CompilerParams(use_tc_tiling_on_sc=True),
        out_type=jax.ShapeDtypeStruct((_B, _L, _W), jnp.float32),
        scratch_types=(
            [pltpu.VMEM((_IDS_PER_W,), jnp.int32),
             pltpu.VMEM((_NBUF, _L, _W), jnp.float32)]
            + [pltpu.SemaphoreType.DMA] * (2 * _NBUF)
        ),
    )
    def lookup(ids_hbm, table_hbm, out_hbm, idx_v, rows_v, *sems):
        gsem = sems[:_NBUF]
        wsem = sems[_NBUF:]
        wid = lax.axis_index("s") * _NC + lax.axis_index("c")
        base = wid * _BATCHES_PER_W
        pltpu.sync_copy(ids_hbm.at[pl.ds(base * _L, _IDS_PER_W)], idx_v)

        def start_gather(i, s):
            pltpu.async_copy(table_hbm.at[idx_v.at[pl.ds(i * _L, _C0)]],
                             rows_v.at[s, pl.ds(0, _C0)], gsem[s])
            pltpu.async_copy(table_hbm.at[idx_v.at[pl.ds(i * _L + _C0, _C1)]],
                             rows_v.at[s, pl.ds(_C0, _C1)], gsem[s])

        def wait_gather(s):
            pltpu.make_async_copy(table_hbm.at[pl.ds(0, _L)], rows_v.at[s],
                                  gsem[s]).wait()

        def start_write(i, s):
            pltpu.async_copy(rows_v.at[s], out_hbm.at[base + i], wsem[s])

        def wait_write(i, s):
            pltpu.make_async_copy(rows_v.at[s], out_hbm.at[base + i],
                                  wsem[s]).wait()

        for s in range(_G):
            start_gather(s, s)

        for s in range(_NBUF):
            i = s
            wait_gather(s)
            start_write(i, s)
            ig = i + _G
            sg = ig % _NBUF
            if ig >= _NBUF:
                wait_write(ig - _NBUF, sg)
            start_gather(ig, sg)

        def block(k, carry):
            i0 = k * _NBUF
            for s in range(_NBUF):
                i = i0 + s
                wait_gather(s)
                start_write(i, s)
                ig = i + _G
                sg = (s + _G) % _NBUF
                wait_write(ig - _NBUF, sg)
                start_gather(ig, sg)
            return carry

        lax.fori_loop(1, _NBLK - 1, block, 0)

        i0 = (_NBLK - 1) * _NBUF
        for s in range(_NBUF):
            i = i0 + s
            wait_gather(s)
            start_write(i, s)
            ig = i + _G
            if ig < _BATCHES_PER_W:
                sg = (s + _G) % _NBUF
                wait_write(ig - _NBUF, sg)
                start_gather(ig, sg)

        for s in range(_NBUF):
            wait_write(i0 + s, s)

    return lookup


_lookup = _make_lookup()


def kernel(word_ids, word_emb_table):
    ids_flat = word_ids.astype(jnp.int32).reshape(-1)
    tpad = jnp.pad(word_emb_table, ((0, 0), (0, _W - _EMB)))
    out = _lookup(ids_flat, tpad)
    return out[:, :, :_EMB]


# speedup vs baseline: 1.3304x; 1.0918x over previous
import functools

import jax
import jax.numpy as jnp
from jax import lax
from jax.experimental import pallas as pl
from jax.experimental.pallas import tpu as pltpu
from jax.experimental.pallas import tpu_sc as plsc

_B = 4096
_L = 200
_EMB = 64
_W = 128
_C0 = 128
_C1 = _L - _C0
_NBUF = 4
_G = 2

_info = plsc.get_sparse_core_info()
_NC, _NS = _info.num_cores, _info.num_subcores
_NW = _NC * _NS
_BATCHES_PER_W = _B // _NW
_IDS_PER_W = _BATCHES_PER_W * _L
_NBLK = _BATCHES_PER_W // _NBUF


def _make_lookup():
    mesh = plsc.VectorSubcoreMesh(core_axis_name="c", subcore_axis_name="s")

    @functools.partial(
        pl.kernel,
        mesh=mesh,
        compiler_params=pltpu.CompilerParams(use_tc_tiling_on_sc=False),
        out_type=jax.ShapeDtypeStruct((_B, _L, _W), jnp.float32),
        scratch_types=(
            [pltpu.VMEM((_IDS_PER_W,), jnp.int32),
             pltpu.VMEM((_NBUF, _L, _EMB), jnp.float32)]
            + [pltpu.SemaphoreType.DMA] * (2 * _NBUF)
        ),
    )
    def lookup(ids_hbm, table_hbm, out_hbm, idx_v, rows_v, *sems):
        gsem = sems[:_NBUF]
        wsem = sems[_NBUF:]
        wid = lax.axis_index("s") * _NC + lax.axis_index("c")
        base = wid * _BATCHES_PER_W
        pltpu.sync_copy(ids_hbm.at[pl.ds(base * _L, _IDS_PER_W)], idx_v)

        def start_gather(i, s):
            pltpu.async_copy(table_hbm.at[idx_v.at[pl.ds(i * _L, _C0)]],
                             rows_v.at[s, pl.ds(0, _C0)], gsem[s])
            pltpu.async_copy(table_hbm.at[idx_v.at[pl.ds(i * _L + _C0, _C1)]],
                             rows_v.at[s, pl.ds(_C0, _C1)], gsem[s])

        def wait_gather(s):
            pltpu.make_async_copy(table_hbm.at[pl.ds(0, _L)], rows_v.at[s],
                                  gsem[s]).wait()

        def start_write(i, s):
            pltpu.async_copy(rows_v.at[s],
                             out_hbm.at[base + i, slice(None), pl.ds(0, _EMB)],
                             wsem[s])

        def wait_write(i, s):
            pltpu.make_async_copy(rows_v.at[s],
                                  out_hbm.at[base + i, slice(None),
                                             pl.ds(0, _EMB)],
                                  wsem[s]).wait()

        for s in range(_G):
            start_gather(s, s)

        for s in range(_NBUF):
            i = s
            wait_gather(s)
            start_write(i, s)
            ig = i + _G
            sg = ig % _NBUF
            if ig >= _NBUF:
                wait_write(ig - _NBUF, sg)
            start_gather(ig, sg)

        def block(k, carry):
            i0 = k * _NBUF
            for s in range(_NBUF):
                i = i0 + s
                wait_gather(s)
                start_write(i, s)
                ig = i + _G
                sg = (s + _G) % _NBUF
                wait_write(ig - _NBUF, sg)
                start_gather(ig, sg)
            return carry

        lax.fori_loop(1, _NBLK - 1, block, 0)

        i0 = (_NBLK - 1) * _NBUF
        for s in range(_NBUF):
            i = i0 + s
            wait_gather(s)
            start_write(i, s)
            ig = i + _G
            if ig < _BATCHES_PER_W:
                sg = (s + _G) % _NBUF
                wait_write(ig - _NBUF, sg)
                start_gather(ig, sg)

        for s in range(_NBUF):
            wait_write(i0 + s, s)

    return lookup


_lookup = _make_lookup()


def kernel(word_ids, word_emb_table):
    ids_flat = word_ids.astype(jnp.int32).reshape(-1)
    out = _lookup(ids_flat, word_emb_table)
    return out[:, :, :_EMB]
